# baseline (speedup 1.0000x reference)
import functools

import jax
import jax.numpy as jnp
from jax import lax
from jax.experimental import pallas as pl
from jax.experimental.pallas import tpu as pltpu
from jax.experimental.pallas import tpu_sc as plsc

N_TOKENS = 9216
N_CODES = 8192
FDIM = 64

BM = 512
BN = 512
NT = N_CODES // BN


_NWIN = 4
_WTILES = NT // _NWIN


def _argmin_body(x2_ref, xsq_ref, embT_ref, esq_ref, idx_ref, rmin_ref, ridx_ref):
    x2 = x2_ref[...]
    xsq = xsq_ref[...]
    wins = []

    for h in range(_NWIN):
        rmin_ref[...] = jnp.full((BM, BN), jnp.inf, jnp.float32)
        ridx_ref[...] = jnp.zeros((BM, BN), jnp.int32)

        def body(t, carry):
            off = h * _WTILES + t
            embT_t = embT_ref[:, pl.ds(off * BN, BN)]
            mm2 = lax.dot_general(x2, embT_t, (((1,), (0,)), ((), ())),
                                  preferred_element_type=jnp.float32)
            d = (xsq + esq_ref[:, pl.ds(off * BN, BN)]) - mm2
            col = lax.broadcasted_iota(jnp.int32, (BM, BN), 1) + off * BN
            rmin = rmin_ref[...]
            mask = d < rmin
            rmin_ref[...] = jnp.where(mask, d, rmin)
            ridx_ref[...] = jnp.where(mask, col, ridx_ref[...])
            return carry

        lax.fori_loop(0, _WTILES, body, 0)

        rmin = rmin_ref[...]
        row_min = jnp.min(rmin, axis=1, keepdims=True)
        cand = jnp.where(rmin == row_min, ridx_ref[...], jnp.int32(2**30))
        wins.append((row_min, jnp.min(cand, axis=1, keepdims=True)))

    ms = wins[0][0].astype(jnp.bfloat16).astype(jnp.float32)
    idx = wins[0][1]
    for w in range(1, _NWIN):
        mw, iw = wins[w]
        pred = mw < ms
        ms = jnp.where(pred, mw.astype(jnp.bfloat16).astype(jnp.float32), ms)
        idx = jnp.where(pred, iw, idx)
    idx_ref[...] = idx


def _argmin_call(x2, xsq, embT, esq):
    return pl.pallas_call(
        _argmin_body,
        grid=(N_TOKENS // BM,),
        in_specs=[
            pl.BlockSpec((BM, FDIM), lambda i: (i, 0)),
            pl.BlockSpec((BM, 1), lambda i: (i, 0)),
            pl.BlockSpec((FDIM, N_CODES), lambda i: (0, 0)),
            pl.BlockSpec((1, N_CODES), lambda i: (0, 0)),
        ],
        out_specs=pl.BlockSpec((BM, 1), lambda i: (i, 0)),
        out_shape=jax.ShapeDtypeStruct((N_TOKENS, 1), jnp.int32),
        scratch_shapes=[
            pltpu.VMEM((BM, BN), jnp.float32),
            pltpu.VMEM((BM, BN), jnp.int32),
        ],
    )(x2, xsq, embT, esq)


_NW = 32
_BPW = N_TOKENS // _NW
_CHUNK = 96
_NCH = _BPW // _CHUNK


_GD = 128


def _gather_body(emb_hbm, idx_hbm, x_hbm, xq_hbm, part_hbm,
                 idx_v, rows_v, x_v, acc_v, sem):
    wid = lax.axis_index("s") * 2 + lax.axis_index("c")
    base = wid * _BPW

    for c in range(_NCH):
        pltpu.sync_copy(idx_hbm.at[pl.ds(base + c * _CHUNK, _CHUNK)],
                        idx_v.at[c])
    descs = [
        pltpu.async_copy(emb_hbm.at[idx_v.at[c]],
                         rows_v.at[pl.ds(c * _CHUNK, _CHUNK)], sem)
        for c in range(_NCH)
    ]
    pltpu.sync_copy(x_hbm.at[pl.ds(base, _BPW)], x_v)
    for d in descs:
        d.wait()
    pltpu.sync_copy(rows_v, xq_hbm.at[pl.ds(base, _BPW)])

    def body(r, acc):
        for c in range(FDIM // 16):
            acc = acc + jnp.abs(rows_v[r, pl.ds(c * 16, 16)]
                                - x_v[r, pl.ds(c * 16, 16)])
        return acc

    acc_v[...] = lax.fori_loop(0, _BPW, body, jnp.zeros((16,), jnp.float32))
    pltpu.sync_copy(acc_v, part_hbm.at[wid])


@functools.partial(jax.jit, static_argnums=())
def _gather_call(emb, idx, x_flat):
    mesh = plsc.VectorSubcoreMesh(core_axis_name="c", subcore_axis_name="s")
    f = pl.kernel(
        _gather_body,
        out_type=[
            jax.ShapeDtypeStruct((N_TOKENS, _GD), jnp.float32),
            jax.ShapeDtypeStruct((_NW, 16), jnp.float32),
        ],
        mesh=mesh,
        scratch_types=[
            pltpu.VMEM((_NCH, _CHUNK), jnp.int32),
            pltpu.VMEM((_BPW, _GD), jnp.float32),
            pltpu.VMEM((_BPW, FDIM), jnp.float32),
            pltpu.VMEM((16,), jnp.float32),
            pltpu.SemaphoreType.DMA,
        ],
    )
    return f(emb, idx, x_flat)


def kernel(x, emb):
    x_size = x.shape
    x_flat = x.reshape(-1, x_size[-1])
    xsq = jnp.sum(x_flat ** 2, axis=1, keepdims=True)
    esq = jnp.sum(emb ** 2, axis=1).reshape(1, N_CODES)
    x2 = x_flat * 2.0
    embT = emb.T

    idx = _argmin_call(x2, xsq, embT, esq).reshape(N_TOKENS)
    emb_p = jnp.pad(emb, ((0, 0), (0, _GD - FDIM)))
    x_q, part = _gather_call(emb_p, idx, x_flat)
    loss = jnp.sum(part) / jnp.float32(N_TOKENS * FDIM)
    return (loss, x_q[:, :FDIM].reshape(x_size))
